# baseline (speedup 1.0000x reference)
import functools

import jax
import jax.numpy as jnp
from jax import lax
from jax.experimental import pallas as pl
from jax.experimental.pallas import tpu as pltpu
from jax.experimental.pallas import tpu_sc as plsc

B = 1024
N = 200
COORD = 128
T = B * N
D_OUT = 6 * COORD
NW = 32
TPW = T // NW
C = 64
NCHUNK = TPW // C
NPAIR = NCHUNK // 2


def _make_sc_kernel():
    mesh = plsc.VectorSubcoreMesh(core_axis_name="c", subcore_axis_name="s")

    @functools.partial(
        pl.kernel,
        out_type=jax.ShapeDtypeStruct((T, D_OUT), jnp.float32),
        mesh=mesh,
        scratch_types=[
            pltpu.VMEM((4, 2 * C), jnp.int32),
            [[pltpu.VMEM((C,), jnp.int32) for _ in range(6)] for _ in range(2)],
            [[pltpu.VMEM((C, COORD), jnp.float32) for _ in range(6)] for _ in range(2)],
            [pltpu.SemaphoreType.DMA for _ in range(2)],
            [pltpu.SemaphoreType.DMA for _ in range(2)],
            [pltpu.VMEM_SHARED((1024, COORD), jnp.float32) for _ in range(3)],
        ],
    )
    def body(bbox_hbm, x_hbm, y_hbm, h_hbm, w_hbm, out_hbm,
             bb_v, idx_v, row_v, gsem, ssem, sp):
        sid = lax.axis_index("s")
        wid = sid * 2 + lax.axis_index("c")
        wbase = wid * TPW

        for t, src in enumerate((x_hbm, y_hbm, h_hbm)):
            stripe = pl.ds(sid * 64, 64)
            pltpu.sync_copy(src.at[stripe, :], row_v[0][0])
            pltpu.sync_copy(row_v[0][0], sp[t].at[stripe, :])
        plsc.subcore_barrier()

        sp_tables = (sp[0], sp[1], sp[0], sp[1], sp[2])

        def out_slice(ci, g):
            return out_hbm.at[pl.ds(wbase + ci * C, C),
                              pl.ds(g * COORD, COORD)]

        def ws(ci, s):
            for g in range(6):
                pltpu.make_async_copy(
                    row_v[s][g], out_slice(ci, g), ssem[s]).wait()

        def process(ci, s, first=False):
            if not first:
                ws(ci - 2, s)
            if s == 0:
                pltpu.sync_copy(
                    bbox_hbm.at[:, pl.ds(wbase + ci * C, 2 * C)], bb_v)
            for i in range(C // 16):
                sl = pl.ds(i * 16, 16)
                bsl = pl.ds(s * C + i * 16, 16)
                c0 = bb_v[0, bsl]
                c1 = bb_v[1, bsl]
                c2 = bb_v[2, bsl]
                c3 = bb_v[3, bsl]
                idx_v[s][0][sl] = c0
                idx_v[s][1][sl] = c1
                idx_v[s][2][sl] = c2
                idx_v[s][3][sl] = c3
                idx_v[s][4][sl] = c3 - c1
                idx_v[s][5][sl] = c2 - c0
            pltpu.async_copy(w_hbm.at[idx_v[s][5]], row_v[s][5], gsem[s])
            for g in range(5):
                pltpu.sync_copy(sp_tables[g].at[idx_v[s][g]], row_v[s][g])
                pltpu.async_copy(row_v[s][g], out_slice(ci, g), ssem[s])
            pltpu.make_async_copy(
                w_hbm.at[idx_v[s][5]], row_v[s][5], gsem[s]).wait()
            pltpu.async_copy(row_v[s][5], out_slice(ci, 5), ssem[s])

        process(0, 0, first=True)
        process(1, 1, first=True)

        def pair(p, _):
            process(2 * p, 0)
            process(2 * p + 1, 1)
            return 0

        lax.fori_loop(1, NPAIR, pair, 0)
        ws(NCHUNK - 2, 0)
        ws(NCHUNK - 1, 1)

    return body


_sc_kernel = _make_sc_kernel()


def kernel(bbox, x_table, y_table, h_table, w_table):
    bbox_t = jnp.transpose(bbox.reshape(T, 4))
    out = _sc_kernel(bbox_t, x_table, y_table, h_table, w_table)
    return out.reshape(B, N, D_OUT)
